# initial kernel scaffold (unmeasured)
import jax
import jax.numpy as jnp
from jax import lax
from jax.experimental import pallas as pl
from jax.experimental.pallas import tpu as pltpu

W = 32
M = 4096
K = 4096
N = 8192
M_PER = M // W
N_PER = N // W


def kernel(x, w_mat, scale_x, scale_w):
    def body(x_ref, w_ref, sx_ref, sw_ref, out_ref,
             stage_ref, send_sems, recv_sems):
        my = lax.axis_index("i")
        scale = sx_ref[0] * sw_ref[0]

        sends = []
        for t in range(1, W):
            j = lax.rem(my + t, W)
            acc = jnp.dot(
                x_ref[...],
                w_ref[:, pl.ds(j * N_PER, N_PER)],
                preferred_element_type=jnp.int32,
            )
            stage_ref[t - 1, :, :] = jnp.maximum(
                acc.astype(jnp.float32) * scale, 0.0
            )
            rdma = pltpu.make_async_remote_copy(
                src_ref=stage_ref.at[t - 1],
                dst_ref=out_ref.at[pl.ds(my * M_PER, M_PER)],
                send_sem=send_sems.at[t],
                recv_sem=recv_sems.at[t],
                device_id=(j,),
                device_id_type=pl.DeviceIdType.MESH,
            )
            rdma.start()
            sends.append(rdma)

        acc = jnp.dot(
            x_ref[...],
            w_ref[:, pl.ds(my * N_PER, N_PER)],
            preferred_element_type=jnp.int32,
        )
        out_ref[pl.ds(my * M_PER, M_PER), :] = jnp.maximum(
            acc.astype(jnp.float32) * scale, 0.0
        )

        for t in range(1, W):
            s = lax.rem(my - t + W, W)
            recv = pltpu.make_async_remote_copy(
                src_ref=stage_ref.at[t - 1],
                dst_ref=out_ref.at[pl.ds(s * M_PER, M_PER)],
                send_sem=send_sems.at[t],
                recv_sem=recv_sems.at[t],
                device_id=(s,),
                device_id_type=pl.DeviceIdType.MESH,
            )
            recv.wait_recv()
        for rdma in sends:
            rdma.wait_send()

    return pl.pallas_call(
        body,
        out_shape=jax.ShapeDtypeStruct((M, N_PER), jnp.float32),
        in_specs=[
            pl.BlockSpec(memory_space=pltpu.VMEM),
            pl.BlockSpec(memory_space=pltpu.VMEM),
            pl.BlockSpec(memory_space=pltpu.SMEM),
            pl.BlockSpec(memory_space=pltpu.SMEM),
        ],
        out_specs=pl.BlockSpec(memory_space=pltpu.VMEM),
        scratch_shapes=[
            pltpu.VMEM((W - 1, M_PER, N_PER), jnp.float32),
            pltpu.SemaphoreType.DMA((W,)),
            pltpu.SemaphoreType.DMA((W,)),
        ],
    )(x, w_mat, scale_x, scale_w)


# baseline (device time: 82157 ns/iter reference)
import jax
import jax.numpy as jnp
from jax import lax
from jax.experimental import pallas as pl
from jax.experimental.pallas import tpu as pltpu

W = 32
M = 4096
K = 4096
N = 8192
M_PER = M // W
N_PER = N // W


def kernel(x, w_mat, scale_x, scale_w):
    def body(x_ref, w_ref, sx_ref, sw_ref, out_ref,
             stage_ref, send_sems, recv_sems):
        my = lax.axis_index("i")
        scale = sx_ref[0] * sw_ref[0]

        sends = []
        for t in range(1, W):
            j = lax.rem(my + t, W)
            acc = jnp.dot(
                x_ref[...],
                w_ref[:, pl.ds(j * N_PER, N_PER)],
                preferred_element_type=jnp.int32,
            )
            stage_ref[t - 1, :, :] = jnp.maximum(
                acc.astype(jnp.float32) * scale, 0.0
            )
            rdma = pltpu.make_async_remote_copy(
                src_ref=stage_ref.at[t - 1],
                dst_ref=out_ref.at[pl.ds(my * M_PER, M_PER)],
                send_sem=send_sems.at[t],
                recv_sem=recv_sems.at[t],
                device_id=(j,),
                device_id_type=pl.DeviceIdType.MESH,
            )
            rdma.start()
            sends.append(rdma)

        acc = jnp.dot(
            x_ref[...],
            w_ref[:, pl.ds(my * N_PER, N_PER)],
            preferred_element_type=jnp.int32,
        )
        out_ref[pl.ds(my * M_PER, M_PER), :] = jnp.maximum(
            acc.astype(jnp.float32) * scale, 0.0
        )

        for t in range(1, W):
            s = lax.rem(my - t + W, W)
            recv = pltpu.make_async_remote_copy(
                src_ref=stage_ref.at[t - 1],
                dst_ref=out_ref.at[pl.ds(s * M_PER, M_PER)],
                send_sem=send_sems.at[t],
                recv_sem=recv_sems.at[t],
                device_id=(s,),
                device_id_type=pl.DeviceIdType.MESH,
            )
            recv.wait_recv()
        for rdma in sends:
            rdma.wait_send()

    return pl.pallas_call(
        body,
        out_shape=jax.ShapeDtypeStruct((M, N_PER), jnp.float32),
        in_specs=[
            pl.BlockSpec(memory_space=pltpu.VMEM),
            pl.BlockSpec(memory_space=pltpu.VMEM),
            pl.BlockSpec(memory_space=pltpu.SMEM),
            pl.BlockSpec(memory_space=pltpu.SMEM),
        ],
        out_specs=pl.BlockSpec(memory_space=pltpu.VMEM),
        scratch_shapes=[
            pltpu.VMEM((W - 1, M_PER, N_PER), jnp.float32),
            pltpu.SemaphoreType.DMA((W,)),
            pltpu.SemaphoreType.DMA((W,)),
        ],
        compiler_params=pltpu.CompilerParams(
            vmem_limit_bytes=64 * 1024 * 1024,
        ),
    )(x, w_mat, scale_x, scale_w)
